# Initial kernel scaffold; baseline (speedup 1.0000x reference)
import jax
import jax.numpy as jnp
from jax.experimental import pallas as pl


def kernel(inputs, text_table, pos_table):
    raise NotImplementedError("write your pallas kernel here")


# speedup vs baseline: 2.3326x; 2.3326x over previous
import functools

import jax
import jax.numpy as jnp
from jax import lax
from jax.experimental import pallas as pl
from jax.experimental.pallas import tpu as pltpu
from jax.experimental.pallas import tpu_sc as plsc

NC = 2
NS = 16
NW = NC * NS
CHUNK = 128


@functools.cache
def _make_sc_kernel(B, L, V, E):
    R = B * L
    RW = R // NW
    NCH = RW // CHUNK
    EV = E // 16

    mesh = plsc.VectorSubcoreMesh(core_axis_name="c", subcore_axis_name="s")

    @functools.partial(
        pl.kernel,
        out_type=jax.ShapeDtypeStruct((R, E), jnp.float32),
        mesh=mesh,
        scratch_types=[
            pltpu.VMEM((L, E), jnp.float32),
            pltpu.VMEM((CHUNK,), jnp.int32),
            pltpu.VMEM((CHUNK, E), jnp.float32),
            pltpu.SemaphoreType.DMA,
        ],
    )
    def k(inp_hbm, tab_hbm, pos_hbm, out_hbm, pos_v, idx_v, rows_v, sem):
        wid = lax.axis_index("s") * NC + lax.axis_index("c")
        base = wid * RW
        pltpu.sync_copy(pos_hbm, pos_v)

        def chunk_body(c, carry):
            s = base + c * CHUNK
            pltpu.sync_copy(inp_hbm.at[pl.ds(s, CHUNK)], idx_v)
            pltpu.async_copy(tab_hbm.at[idx_v], rows_v, sem).wait()
            off = lax.rem(c * CHUNK, L)

            def row_body(j, carry2):
                p = off + j
                p = jnp.where(p >= L, p - L, p)
                for e in range(EV):
                    plsc.addupdate(rows_v.at[j, pl.ds(16 * e, 16)],
                                   pos_v[p, pl.ds(16 * e, 16)])
                return carry2

            lax.fori_loop(0, CHUNK, row_body, 0)
            pltpu.sync_copy(rows_v, out_hbm.at[pl.ds(s, CHUNK)])
            return carry

        lax.fori_loop(0, NCH, chunk_body, 0)

    return k


def kernel(inputs, text_table, pos_table):
    B, L = inputs.shape
    V, E = text_table.shape
    k = _make_sc_kernel(B, L, V, E)
    flat_idx = inputs.reshape(B * L).astype(jnp.int32)
    out = k(flat_idx, text_table, pos_table)
    return out.reshape(B, L, E)


# speedup vs baseline: 3.3222x; 1.4243x over previous
import functools

import jax
import jax.numpy as jnp
from jax import lax
from jax.experimental import pallas as pl
from jax.experimental.pallas import tpu as pltpu
from jax.experimental.pallas import tpu_sc as plsc

NC = 2
NS = 16
NW = NC * NS
CHUNK = 128
NBUF = 4


@functools.cache
def _make_sc_kernel(B, L, V, E):
    R = B * L
    RW = R // NW
    NCH = RW // CHUNK
    EV = E // 16

    mesh = plsc.VectorSubcoreMesh(core_axis_name="c", subcore_axis_name="s")

    @functools.partial(
        pl.kernel,
        out_type=jax.ShapeDtypeStruct((R, E), jnp.float32),
        mesh=mesh,
        scratch_types=[
            pltpu.VMEM((L, E), jnp.float32),
            pltpu.VMEM((NCH, CHUNK), jnp.int32),
            [pltpu.VMEM((CHUNK, E), jnp.float32) for _ in range(NBUF)],
            [pltpu.SemaphoreType.DMA for _ in range(NBUF)],
            [pltpu.SemaphoreType.DMA for _ in range(NBUF)],
        ],
    )
    def k(inp_hbm, tab_hbm, pos_hbm, out_hbm, pos_v, idx_v, rows, gsem, osem):
        wid = lax.axis_index("s") * NC + lax.axis_index("c")
        base = wid * RW
        pltpu.sync_copy(pos_hbm, pos_v)
        pltpu.sync_copy(inp_hbm.at[wid], idx_v)

        def start_gather(c, b):
            pltpu.async_copy(tab_hbm.at[idx_v.at[c]], rows[b], gsem[b])

        def wait_gather(b):
            pltpu.make_async_copy(tab_hbm.at[idx_v.at[0]], rows[b],
                                  gsem[b]).wait()

        def wait_write(b):
            pltpu.make_async_copy(rows[b], out_hbm.at[pl.ds(0, CHUNK)],
                                  osem[b]).wait()

        for b in range(NBUF - 1):
            start_gather(b, b)

        def step_body(p, carry):
            for b in range(NBUF):
                c = p * NBUF + b
                nxt = (b + NBUF - 1) % NBUF

                @pl.when(c + NBUF - 1 < NCH)
                def _():
                    @pl.when(c > 0)
                    def _():
                        wait_write(nxt)
                    start_gather(c + NBUF - 1, nxt)

                wait_gather(b)
                off = lax.rem(c * CHUNK, L)

                def row_body(j, carry2):
                    q = off + j
                    q = jnp.where(q >= L, q - L, q)
                    for e in range(EV):
                        plsc.addupdate(rows[b].at[j, pl.ds(16 * e, 16)],
                                       pos_v[q, pl.ds(16 * e, 16)])
                    return carry2

                lax.fori_loop(0, CHUNK, row_body, 0)
                pltpu.async_copy(rows[b],
                                 out_hbm.at[pl.ds(base + c * CHUNK, CHUNK)],
                                 osem[b])
            return carry

        lax.fori_loop(0, NCH // NBUF, step_body, 0)
        for b in range(NBUF):
            wait_write(b)

    return k


def kernel(inputs, text_table, pos_table):
    B, L = inputs.shape
    V, E = text_table.shape
    k = _make_sc_kernel(B, L, V, E)
    idx = inputs.reshape(NW, (B * L) // (NW * CHUNK), CHUNK).astype(jnp.int32)
    out = k(idx, text_table, pos_table)
    return out.reshape(B, L, E)


# speedup vs baseline: 7.3663x; 2.2173x over previous
import functools

import jax
import jax.numpy as jnp
from jax import lax
from jax.experimental import pallas as pl
from jax.experimental.pallas import tpu as pltpu
from jax.experimental.pallas import tpu_sc as plsc

NC = 2
NS = 16
NW = NC * NS
CHUNK = 128
NBUF = 4


@functools.cache
def _make_sc_kernel(B, L, V, E):
    R = B * L
    RW = R // NW
    NCH = RW // CHUNK
    EV = E // 16

    mesh = plsc.VectorSubcoreMesh(core_axis_name="c", subcore_axis_name="s")

    @functools.partial(
        pl.kernel,
        out_type=jax.ShapeDtypeStruct((R, E), jnp.float32),
        mesh=mesh,
        scratch_types=[
            pltpu.VMEM((L, E), jnp.float32),
            pltpu.VMEM((NCH, CHUNK), jnp.int32),
            [pltpu.VMEM((CHUNK, E), jnp.float32) for _ in range(NBUF)],
            [pltpu.SemaphoreType.DMA for _ in range(NBUF)],
            [pltpu.SemaphoreType.DMA for _ in range(NBUF)],
        ],
    )
    def k(inp_hbm, tab_hbm, pos_hbm, out_hbm, pos_v, idx_v, rows, gsem, osem):
        wid = lax.axis_index("s") * NC + lax.axis_index("c")
        base = wid * RW
        pltpu.sync_copy(pos_hbm, pos_v)
        pltpu.sync_copy(inp_hbm.at[wid], idx_v)

        def start_gather(c, b):
            pltpu.async_copy(tab_hbm.at[idx_v.at[c]], rows[b], gsem[b])

        def wait_gather(b):
            pltpu.make_async_copy(tab_hbm.at[idx_v.at[0]], rows[b],
                                  gsem[b]).wait()

        def wait_write(b):
            pltpu.make_async_copy(rows[b], out_hbm.at[pl.ds(0, CHUNK)],
                                  osem[b]).wait()

        for b in range(NBUF - 1):
            start_gather(b, b)

        def step_body(p, carry):
            for b in range(NBUF):
                c = p * NBUF + b
                nxt = (b + NBUF - 1) % NBUF

                @pl.when(c + NBUF - 1 < NCH)
                def _():
                    @pl.when(c > 0)
                    def _():
                        wait_write(nxt)
                    start_gather(c + NBUF - 1, nxt)

                wait_gather(b)
                off = lax.rem(c * CHUNK, L)
                n1 = jnp.minimum(CHUNK, L - off)

                @plsc.parallel_loop(0, n1, unroll=4)
                def _(j):
                    for e in range(EV):
                        plsc.addupdate(rows[b].at[j, pl.ds(16 * e, 16)],
                                       pos_v[off + j, pl.ds(16 * e, 16)])

                @plsc.parallel_loop(n1, CHUNK, unroll=4)
                def _(j):
                    for e in range(EV):
                        plsc.addupdate(rows[b].at[j, pl.ds(16 * e, 16)],
                                       pos_v[j - n1, pl.ds(16 * e, 16)])
                pltpu.async_copy(rows[b],
                                 out_hbm.at[pl.ds(base + c * CHUNK, CHUNK)],
                                 osem[b])
            return carry

        lax.fori_loop(0, NCH // NBUF, step_body, 0)
        for b in range(NBUF):
            wait_write(b)

    return k


def kernel(inputs, text_table, pos_table):
    B, L = inputs.shape
    V, E = text_table.shape
    k = _make_sc_kernel(B, L, V, E)
    idx = inputs.reshape(NW, (B * L) // (NW * CHUNK), CHUNK).astype(jnp.int32)
    out = k(idx, text_table, pos_table)
    return out.reshape(B, L, E)


# speedup vs baseline: 8.9972x; 1.2214x over previous
import functools

import jax
import jax.numpy as jnp
from jax import lax
from jax.experimental import pallas as pl
from jax.experimental.pallas import tpu as pltpu
from jax.experimental.pallas import tpu_sc as plsc

NC = 2
NS = 16
NW = NC * NS
HALF = 100
HPAD = 128
NBUF = 4


@functools.cache
def _make_sc_kernel(B, L, V, E):
    R = B * L
    RW = R // NW
    NCH = RW // L
    mesh = plsc.VectorSubcoreMesh(core_axis_name="c", subcore_axis_name="s")

    @functools.partial(
        pl.kernel,
        out_type=jax.ShapeDtypeStruct((R, E), jnp.float32),
        mesh=mesh,
        scratch_types=[
            pltpu.VMEM_SHARED((L, E), jnp.float32),
            [pltpu.VMEM((2, HPAD), jnp.int32) for _ in range(NBUF)],
            [pltpu.VMEM((L, E), jnp.float32) for _ in range(NBUF)],
            [pltpu.SemaphoreType.DMA for _ in range(NBUF)],
            [pltpu.SemaphoreType.DMA for _ in range(NBUF)],
            [pltpu.SemaphoreType.DMA for _ in range(NBUF)],
            [pltpu.SemaphoreType.DMA for _ in range(NBUF)],
        ],
    )
    def k(inp_hbm, tab_hbm, pos_hbm, out_hbm,
          pos_s, ibuf, rows, isem, psem, gsem, osem):
        wid = lax.axis_index("s") * NC + lax.axis_index("c")
        base = wid * RW

        @pl.when(lax.axis_index("s") == 0)
        def _():
            pltpu.sync_copy(pos_hbm, pos_s)
        plsc.subcore_barrier()

        def start_stage(c, b):
            pltpu.async_copy(inp_hbm.at[wid, c], ibuf[b], isem[b])
            pltpu.async_copy(pos_s, rows[b], psem[b])

        def wait_stage(b):
            pltpu.make_async_copy(inp_hbm.at[0, 0], ibuf[b], isem[b]).wait()
            pltpu.make_async_copy(pos_s, rows[b], psem[b]).wait()

        def start_gather(c, b):
            for h in range(2):
                pltpu.async_copy(tab_hbm.at[ibuf[b].at[h, pl.ds(0, HALF)]],
                                 rows[b].at[pl.ds(h * HALF, HALF)],
                                 gsem[b], add=True)

        def wait_gather(b):
            for h in range(2):
                pltpu.make_async_copy(
                    tab_hbm.at[ibuf[b].at[0, pl.ds(0, HALF)]],
                    rows[b].at[pl.ds(h * HALF, HALF)], gsem[b]).wait()

        def wait_write(b):
            pltpu.make_async_copy(rows[b], out_hbm.at[pl.ds(0, L)],
                                  osem[b]).wait()

        for b in range(NBUF - 1):
            start_stage(b, b)
        wait_stage(0)
        start_gather(0, 0)

        def step_body(p, carry):
            for b in range(NBUF):
                c = p * NBUF + b
                b1 = (b + 1) % NBUF
                b3 = (b + NBUF - 1) % NBUF

                @pl.when(c + NBUF - 1 < NCH)
                def _():
                    @pl.when(c > 0)
                    def _():
                        wait_write(b3)
                    start_stage(c + NBUF - 1, b3)

                @pl.when(c + 1 < NCH)
                def _():
                    wait_stage(b1)
                    start_gather(c + 1, b1)

                wait_gather(b)
                pltpu.async_copy(rows[b],
                                 out_hbm.at[pl.ds(base + c * L, L)],
                                 osem[b])
            return carry

        lax.fori_loop(0, NCH // NBUF, step_body, 0)
        for b in range(NBUF):
            wait_write(b)

    return k


def kernel(inputs, text_table, pos_table):
    B, L = inputs.shape
    V, E = text_table.shape
    k = _make_sc_kernel(B, L, V, E)
    NCH = B // NW
    idx = inputs.reshape(NW, NCH, 2, HALF).astype(jnp.int32)
    idx = jnp.pad(idx, ((0, 0), (0, 0), (0, 0), (0, HPAD - HALF)))
    out = k(idx, text_table, pos_table)
    return out.reshape(B, L, E)
